# baseline (speedup 1.0000x reference)
import functools

import jax
import jax.numpy as jnp
from jax import lax
from jax.experimental import pallas as pl
from jax.experimental.pallas import tpu as pltpu
from jax.experimental.pallas import tpu_sc as plsc

BATCH = 16384
EMB = 64
K = 128
W = 128

NC = 2
NS = 16
NW = NC * NS
BPW = BATCH // NW
CH = 128
NCH = BPW // CH


def _sc_gather_body(uid_hbm, iid_hbm, tabu_hbm, tabi_hbm,
                    o_u, o_i,
                    idx_u, idx_i, buf_a, buf_b, sem):
    wid = lax.axis_index("s") * NC + lax.axis_index("c")
    pltpu.sync_copy(uid_hbm.at[wid], idx_u)
    pltpu.sync_copy(iid_hbm.at[wid], idx_i)

    half_ch = NCH // 2
    for half in range(2):
        handles = []
        for j in range(half_ch):
            jj = half * half_ch + j
            handles.append(pltpu.async_copy(tabu_hbm.at[idx_u.at[jj]],
                                            buf_a.at[pl.ds(j * CH, CH)], sem))
            handles.append(pltpu.async_copy(tabi_hbm.at[idx_i.at[jj]],
                                            buf_b.at[pl.ds(j * CH, CH)], sem))
        for h in handles:
            h.wait()
        off = half * half_ch * CH
        pltpu.sync_copy(buf_a, o_u.at[wid, pl.ds(off, half_ch * CH)])
        pltpu.sync_copy(buf_b, o_i.at[wid, pl.ds(off, half_ch * CH)])


_sc_gather = functools.partial(
    pl.kernel,
    mesh=plsc.VectorSubcoreMesh(core_axis_name="c", subcore_axis_name="s"),
    out_type=tuple(
        jax.ShapeDtypeStruct((NW, BPW, W), jnp.float32) for _ in range(2)),
    scratch_types=[
        pltpu.VMEM((NCH, CH), jnp.int32),
        pltpu.VMEM((NCH, CH), jnp.int32),
        pltpu.VMEM((BPW // 2, W), jnp.float32),
        pltpu.VMEM((BPW // 2, W), jnp.float32),
        pltpu.SemaphoreType.DMA,
    ],
)(_sc_gather_body)


def _tc_transpose_body(a_ref, b_ref, out_ref):
    out_ref[...] = jnp.concatenate(
        [a_ref[...].T, b_ref[...].T], axis=1)


def _make_transpose(n_rows, cw=512):
    n_steps = -(-n_rows // cw)
    n_out = n_steps * cw
    return pl.pallas_call(
        _tc_transpose_body,
        grid=(n_steps,),
        in_specs=[
            pl.BlockSpec((EMB, cw), lambda i: (0, i)),
            pl.BlockSpec((EMB, cw), lambda i: (0, i)),
        ],
        out_specs=pl.BlockSpec((cw, W), lambda i: (i, 0)),
        out_shape=jax.ShapeDtypeStruct((n_out, W), jnp.float32),
    )


def _tc_mlp_body(ru_ref, ri_ref, w1a_ref, w1b_ref, b1_ref,
                 w2_ref, b2_ref, wa_ref, wb_ref, bout_ref, out_ref):
    f32 = jnp.float32
    ru = ru_ref[...]
    ri = ri_ref[...]
    h = jnp.dot(ru, w1a_ref[...], preferred_element_type=f32)
    h += jnp.dot(ri, w1b_ref[...], preferred_element_type=f32)
    h = jnp.maximum(h + b1_ref[...], 0.0)
    h = jnp.dot(h, w2_ref[...], preferred_element_type=f32)
    h = jnp.maximum(h + b2_ref[...], 0.0)
    xmf = ru * ri
    z = jnp.dot(xmf, wa_ref[...], preferred_element_type=f32)
    z += jnp.dot(h, wb_ref[...], preferred_element_type=f32)
    z += bout_ref[...]
    out_ref[...] = jnp.where(z > 0.0, z, jnp.exp(z) - 1.0)


def kernel(user_id, item_id, mf_user, mf_item, mlp_user, mlp_item,
           W1, b1, W2, b2, Wout, bout):
    uid = user_id.astype(jnp.int32).reshape(NW, NCH, CH)
    iid = item_id.astype(jnp.int32).reshape(NW, NCH, CH)
    tab_u = _make_transpose(mf_user.shape[0])(mf_user.T, mlp_user.T)
    tab_i = _make_transpose(mf_item.shape[0])(mf_item.T, mlp_item.T)
    ru, ri = _sc_gather(uid, iid, tab_u, tab_i)
    ru = ru.reshape(BATCH, W)
    ri = ri.reshape(BATCH, W)

    zpad = jnp.zeros((EMB, K), jnp.float32)
    w1a = jnp.concatenate([zpad, W1[:EMB, :]], axis=0)
    w1b = jnp.concatenate([zpad, W1[EMB:, :]], axis=0)
    wa = jnp.concatenate([Wout[:EMB, :], jnp.zeros((EMB, 1), jnp.float32)],
                         axis=0)
    wb = Wout[EMB:, :]

    BLK = 2048
    grid = (BATCH // BLK,)
    zero = lambda i: (0, 0)
    out = pl.pallas_call(
        _tc_mlp_body,
        grid=grid,
        in_specs=[
            pl.BlockSpec((BLK, W), lambda i: (i, 0)),
            pl.BlockSpec((BLK, W), lambda i: (i, 0)),
            pl.BlockSpec((W, K), zero),
            pl.BlockSpec((W, K), zero),
            pl.BlockSpec((1, K), zero),
            pl.BlockSpec((K, K), zero),
            pl.BlockSpec((1, K), zero),
            pl.BlockSpec((W, 1), zero),
            pl.BlockSpec((K, 1), zero),
            pl.BlockSpec((1, 1), zero),
        ],
        out_specs=pl.BlockSpec((BLK, 1), lambda i: (i, 0)),
        out_shape=jax.ShapeDtypeStruct((BATCH, 1), jnp.float32),
    )(
        ru, ri,
        w1a, w1b, b1.reshape(1, K),
        W2, b2.reshape(1, K),
        wa, wb, bout.reshape(1, 1),
    )
    return out


# speedup vs baseline: 1.6968x; 1.6968x over previous
import functools

import jax
import jax.numpy as jnp
from jax import lax
from jax.experimental import pallas as pl
from jax.experimental.pallas import tpu as pltpu
from jax.experimental.pallas import tpu_sc as plsc

BATCH = 16384
EMB = 64
K = 128

NC = 2
NS = 16
NW = NC * NS
BPW = BATCH // NW
CH = 128
NCH = BPW // CH
GC = 32
NCK = BPW // GC
L = 16


def _issue_chunk(tab, idx, gbuf, j, off, sem):
    handles = []
    for v in range(GC // L):
        rv = idx[j, pl.ds(off + v * L, L)]
        gv = lax.bitwise_and(rv, jnp.int32(-8))
        for k in range(L):
            base = pl.multiple_of(gv[k], 8)
            handles.append(pltpu.async_copy(
                tab.at[pl.ds(base, 8)], gbuf.at[v * L + k], sem))
    return handles


def _extract_rows(idx, gbuf, mini, j, off):
    for v in range(GC // L):
        rv = idx[j, pl.ds(off + v * L, L)]
        r8v = lax.rem(rv, 8)
        for k in range(L):
            r8 = r8v[k]
            row = v * L + k
            for c in range(EMB // L):
                sl = pl.ds(c * L, L)
                mini[row, sl] = gbuf[row, r8, sl]


def _sc_gather_body(uid_hbm, iid_hbm, mfu_hbm, mfi_hbm, mlpu_hbm, mlpi_hbm,
                    o_mfu, o_mfi, o_u, o_i,
                    idx_u, idx_i, gbuf_a, gbuf_b, mini, sem):
    wid = lax.axis_index("s") * NC + lax.axis_index("c")
    pltpu.sync_copy(uid_hbm.at[wid], idx_u)
    pltpu.sync_copy(iid_hbm.at[wid], idx_i)

    for tab, idx, out in (
        (mfu_hbm, idx_u, o_mfu),
        (mfi_hbm, idx_i, o_mfi),
        (mlpu_hbm, idx_u, o_u),
        (mlpi_hbm, idx_i, o_i),
    ):
        def pair(p, carry, tab=tab, idx=idx, out=out):
            ca = 2 * p
            cb = 2 * p + 1
            ja = lax.div(ca, NCK // NCH)
            oa = lax.rem(ca, NCK // NCH) * GC
            jb = lax.div(cb, NCK // NCH)
            ob = lax.rem(cb, NCK // NCH) * GC
            ha = _issue_chunk(tab, idx, gbuf_a, ja, oa, sem)
            hb = _issue_chunk(tab, idx, gbuf_b, jb, ob, sem)
            for h_ in ha:
                h_.wait()
            _extract_rows(idx, gbuf_a, mini, ja, oa)
            pltpu.sync_copy(
                mini, out.at[wid, pl.ds(pl.multiple_of(ca * GC, GC), GC)])
            for h_ in hb:
                h_.wait()
            _extract_rows(idx, gbuf_b, mini, jb, ob)
            pltpu.sync_copy(
                mini, out.at[wid, pl.ds(pl.multiple_of(cb * GC, GC), GC)])
            return carry
        lax.fori_loop(0, NCK // 2, pair, 0)


_sc_gather = functools.partial(
    pl.kernel,
    mesh=plsc.VectorSubcoreMesh(core_axis_name="c", subcore_axis_name="s"),
    out_type=tuple(
        jax.ShapeDtypeStruct((NW, BPW, EMB), jnp.float32) for _ in range(4)),
    scratch_types=[
        pltpu.VMEM((NCH, CH), jnp.int32),
        pltpu.VMEM((NCH, CH), jnp.int32),
        pltpu.VMEM((GC, 8, EMB), jnp.float32),
        pltpu.VMEM((GC, 8, EMB), jnp.float32),
        pltpu.VMEM((GC, EMB), jnp.float32),
        pltpu.SemaphoreType.DMA,
    ],
)(_sc_gather_body)


def _tc_mlp_body(mfu_ref, mfi_ref, xu_ref, xi_ref, w1a_ref, w1b_ref, b1_ref,
                 w2_ref, b2_ref, wa_ref, wb_ref, bout_ref, out_ref):
    f32 = jnp.float32
    h = jnp.dot(xu_ref[...], w1a_ref[...], preferred_element_type=f32)
    h += jnp.dot(xi_ref[...], w1b_ref[...], preferred_element_type=f32)
    h = jnp.maximum(h + b1_ref[...], 0.0)
    h = jnp.dot(h, w2_ref[...], preferred_element_type=f32)
    h = jnp.maximum(h + b2_ref[...], 0.0)
    xmf = mfu_ref[...] * mfi_ref[...]
    z = jnp.dot(xmf, wa_ref[...], preferred_element_type=f32)
    z += jnp.dot(h, wb_ref[...], preferred_element_type=f32)
    z += bout_ref[...]
    out_ref[...] = jnp.where(z > 0.0, z, jnp.exp(z) - 1.0)


def kernel(user_id, item_id, mf_user, mf_item, mlp_user, mlp_item,
           W1, b1, W2, b2, Wout, bout):
    uid = user_id.astype(jnp.int32).reshape(NW, NCH, CH)
    iid = item_id.astype(jnp.int32).reshape(NW, NCH, CH)
    mfu, mfi, xu, xi = _sc_gather(
        uid, iid,
        mf_user[:1000000], mf_item[:100000],
        mlp_user[:1000000], mlp_item[:100000])
    mfu = mfu.reshape(BATCH, EMB)
    mfi = mfi.reshape(BATCH, EMB)
    xu = xu.reshape(BATCH, EMB)
    xi = xi.reshape(BATCH, EMB)

    BLK = 2048
    grid = (BATCH // BLK,)
    zero = lambda i: (0, 0)
    out = pl.pallas_call(
        _tc_mlp_body,
        grid=grid,
        in_specs=[
            pl.BlockSpec((BLK, EMB), lambda i: (i, 0)),
            pl.BlockSpec((BLK, EMB), lambda i: (i, 0)),
            pl.BlockSpec((BLK, EMB), lambda i: (i, 0)),
            pl.BlockSpec((BLK, EMB), lambda i: (i, 0)),
            pl.BlockSpec((EMB, K), zero),
            pl.BlockSpec((EMB, K), zero),
            pl.BlockSpec((1, K), zero),
            pl.BlockSpec((K, K), zero),
            pl.BlockSpec((1, K), zero),
            pl.BlockSpec((EMB, 1), zero),
            pl.BlockSpec((K, 1), zero),
            pl.BlockSpec((1, 1), zero),
        ],
        out_specs=pl.BlockSpec((BLK, 1), lambda i: (i, 0)),
        out_shape=jax.ShapeDtypeStruct((BATCH, 1), jnp.float32),
    )(
        mfu, mfi, xu, xi,
        W1[:EMB, :], W1[EMB:, :], b1.reshape(1, K),
        W2, b2.reshape(1, K),
        Wout[:EMB, :], Wout[EMB:, :], bout.reshape(1, 1),
    )
    return out
